# baseline (speedup 1.0000x reference)
import functools

import jax
import jax.numpy as jnp
from jax import lax
from jax.experimental import pallas as pl
from jax.experimental.pallas import tpu as pltpu
from jax.experimental.pallas import tpu_sc as plsc

N = 10000
E = 320000
F = 128
D = 128

NC = 2
NS = 16
K = 128
ER = E // K
NPAD = 10112
SL = NPAD // NS
_CP = (SL // K) * K


_ROWS1 = ER // NS
_EXTRA1 = ER - _ROWS1 * NS
_FD = 12


def _deg_body(ed_hbm, zvec_hbm, deg_out, idxb, ones_v, zv, deg_sh, sem):
    sid = lax.axis_index("s")

    pltpu.sync_copy(zvec_hbm, zv)
    pltpu.sync_copy(zv, deg_sh.at[pl.ds(sid * SL, SL)])

    for i in range(K // 16):
        ones_v[pl.ds(i * 16, 16)] = jnp.ones((16,), jnp.float32)

    pltpu.sync_copy(ed_hbm.at[1, pl.ds(sid * _ROWS1, _ROWS1)],
                    idxb.at[pl.ds(0, _ROWS1)])

    @pl.when(sid < _EXTRA1 // 2)
    def _():
        pltpu.sync_copy(ed_hbm.at[1, pl.ds(ER - _EXTRA1 + 2 * sid, 2)],
                        idxb.at[pl.ds(_ROWS1, 2)])

    plsc.subcore_barrier()

    def blk(t, _):
        descs = [
            pltpu.async_copy(ones_v, deg_sh.at[idxb.at[t * _FD + j, 0]], sem,
                             add=True)
            for j in range(_FD)
        ]
        for d in descs:
            d.wait()
        return ()

    lax.fori_loop(0, _ROWS1 // _FD, blk, (), unroll=False)

    @pl.when(sid < _EXTRA1 // 2)
    def _():
        pltpu.sync_copy(ones_v, deg_sh.at[idxb.at[_ROWS1, 0]], add=True)
        pltpu.sync_copy(ones_v, deg_sh.at[idxb.at[_ROWS1 + 1, 0]], add=True)

    plsc.subcore_barrier()

    pltpu.sync_copy(deg_sh.at[pl.ds(sid * SL, SL)], zv)
    pltpu.sync_copy(zv, deg_out.at[pl.ds(sid * SL, SL)])


@functools.cache
def _deg_kernel():
    return functools.partial(
        pl.kernel,
        out_type=jax.ShapeDtypeStruct((NPAD,), jnp.float32),
        mesh=plsc.VectorSubcoreMesh(
            core_axis_name="c", subcore_axis_name="s", num_cores=1),
        scratch_types=[
            pltpu.VMEM((_ROWS1 + 2, 1, K), jnp.int32),
            pltpu.VMEM((K,), jnp.float32),
            pltpu.VMEM((SL,), jnp.float32),
            pltpu.MemorySpace.VMEM_SHARED((NPAD,), jnp.float32),
            pltpu.SemaphoreType.DMA,
        ],
    )(_deg_body)


NW = NC * NS
_ROWS2 = ER // NW
_EXTRA2 = ER - _ROWS2 * NW
_BR = 26
_NB = _ROWS2 // _BR
_BP = _BR // 2


def _agg_body(ed_hbm, feat_hbm, deg_hbm, zvec_hbm, zrows_hbm,
              agg_out, c_out, idxs0, idxd0, idxs1, idxd1, idxsx, idxdx,
              rows0, rows1, degv0, degv1, wv0, wv1, zv, agg_sh, c_sh,
              gsem0, gsem1, dsem0, dsem1, ssem0, ssem1, csem0, csem1,
              ibsem0, ibsem1):
    cid = lax.axis_index("c")
    sid = lax.axis_index("s")
    wid = sid * NC + cid
    base = wid * _ROWS2

    pltpu.sync_copy(zrows_hbm, rows0)
    for k in range(SL // K):
        pltpu.sync_copy(rows0, agg_sh.at[pl.ds(sid * SL + k * K, K)])
    pltpu.sync_copy(rows0.at[pl.ds(0, SL - _CP)],
                    agg_sh.at[pl.ds(sid * SL + _CP, SL - _CP)])
    pltpu.sync_copy(zvec_hbm, zv)
    pltpu.sync_copy(zv, c_sh.at[pl.ds(sid * SL, SL)])

    pltpu.sync_copy(ed_hbm.at[0, pl.ds(base, _BR)], idxs0)
    pltpu.sync_copy(ed_hbm.at[1, pl.ds(base, _BR)], idxd0)
    pltpu.async_copy(ed_hbm.at[0, pl.ds(base + _BR, _BR)], idxs1, ibsem1)
    pltpu.async_copy(ed_hbm.at[1, pl.ds(base + _BR, _BR)], idxd1, ibsem1)

    @pl.when(wid < _EXTRA2 // 2)
    def _():
        pltpu.sync_copy(ed_hbm.at[0, pl.ds(ER - _EXTRA2 + 2 * wid, 2)], idxsx)
        pltpu.sync_copy(ed_hbm.at[1, pl.ds(ER - _EXTRA2 + 2 * wid, 2)], idxdx)

    plsc.subcore_barrier()

    def recip(degv, wv):
        for i in range(K // 16):
            dv = degv[pl.ds(i * 16, 16)]
            wv[pl.ds(i * 16, 16)] = 1.0 / jnp.maximum(dv, 1.0)

    def wait_g(rows_v, gsem):
        pltpu.make_async_copy(feat_hbm.at[pl.ds(0, K)], rows_v, gsem).wait()

    def wait_d(degv, dsem):
        pltpu.make_async_copy(deg_hbm.at[pl.ds(0, K)], degv, dsem).wait()

    def wait_s(rows_v, ssem):
        pltpu.make_async_copy(rows_v, agg_sh.at[pl.ds(0, K)], ssem).wait()

    def wait_c(wv, csem):
        pltpu.make_async_copy(wv, c_sh.at[pl.ds(0, K)], csem).wait()

    def wait_ib(idxs_b, idxd_b, ibsem):
        pltpu.make_async_copy(ed_hbm.at[0, pl.ds(0, _BR)], idxs_b, ibsem).wait()
        pltpu.make_async_copy(ed_hbm.at[1, pl.ds(0, _BR)], idxd_b, ibsem).wait()

    def run_block(idxs_b, idxd_b):
        def g_start(slot, rows_v, gsem, degv, dsem):
            pltpu.async_copy(feat_hbm.at[idxs_b.at[slot, 0]], rows_v, gsem)
            pltpu.async_copy(deg_hbm.at[idxd_b.at[slot, 0]], degv, dsem)

        g_start(0, rows0, gsem0, degv0, dsem0)

        def step(t, _):
            a = 2 * t
            b = a + 1
            wait_g(rows0, gsem0)

            @pl.when(t > 0)
            def _():
                wait_s(rows1, ssem1)

            g_start(b, rows1, gsem1, degv1, dsem1)
            pltpu.async_copy(rows0, agg_sh.at[idxd_b.at[a, 0]], ssem0,
                             add=True)
            wait_d(degv0, dsem0)

            @pl.when(t > 0)
            def _():
                wait_c(wv0, csem0)

            recip(degv0, wv0)
            pltpu.async_copy(wv0, c_sh.at[idxs_b.at[a, 0]], csem0, add=True)

            wait_g(rows1, gsem1)
            wait_s(rows0, ssem0)

            @pl.when(t < _BP - 1)
            def _():
                g_start(a + 2, rows0, gsem0, degv0, dsem0)

            pltpu.async_copy(rows1, agg_sh.at[idxd_b.at[b, 0]], ssem1,
                             add=True)
            wait_d(degv1, dsem1)

            @pl.when(t > 0)
            def _():
                wait_c(wv1, csem1)

            recip(degv1, wv1)
            pltpu.async_copy(wv1, c_sh.at[idxs_b.at[b, 0]], csem1, add=True)
            return ()

        lax.fori_loop(0, _BP, step, (), unroll=False)
        wait_s(rows1, ssem1)
        wait_c(wv0, csem0)
        wait_c(wv1, csem1)

    run_block(idxs0, idxd0)
    pltpu.async_copy(ed_hbm.at[0, pl.ds(base + 2 * _BR, _BR)], idxs0, ibsem0)
    pltpu.async_copy(ed_hbm.at[1, pl.ds(base + 2 * _BR, _BR)], idxd0, ibsem0)
    wait_ib(idxs1, idxd1, ibsem1)
    run_block(idxs1, idxd1)
    wait_ib(idxs0, idxd0, ibsem0)
    run_block(idxs0, idxd0)

    @pl.when(wid < _EXTRA2 // 2)
    def _():
        for r in range(2):
            pltpu.async_copy(feat_hbm.at[idxsx.at[r, 0]], rows0, gsem0)
            pltpu.async_copy(deg_hbm.at[idxdx.at[r, 0]], degv0, dsem0)
            wait_g(rows0, gsem0)
            pltpu.sync_copy(rows0, agg_sh.at[idxdx.at[r, 0]], add=True)
            wait_d(degv0, dsem0)
            recip(degv0, wv0)
            pltpu.sync_copy(wv0, c_sh.at[idxsx.at[r, 0]], add=True)

    plsc.subcore_barrier()

    for k in range(SL // K):
        pltpu.sync_copy(agg_sh.at[pl.ds(sid * SL + k * K, K)], rows0)
        pltpu.sync_copy(rows0, agg_out.at[cid, pl.ds(sid * SL + k * K, K)])
    pltpu.sync_copy(agg_sh.at[pl.ds(sid * SL + _CP, SL - _CP)],
                    rows0.at[pl.ds(0, SL - _CP)])
    pltpu.sync_copy(rows0.at[pl.ds(0, SL - _CP)],
                    agg_out.at[cid, pl.ds(sid * SL + _CP, SL - _CP)])
    pltpu.sync_copy(c_sh.at[pl.ds(sid * SL, SL)], zv)
    pltpu.sync_copy(zv, c_out.at[pl.ds(cid * NPAD + sid * SL, SL)])


@functools.cache
def _agg_kernel():
    return functools.partial(
        pl.kernel,
        out_type=[
            jax.ShapeDtypeStruct((NC, NPAD, F), jnp.float32),
            jax.ShapeDtypeStruct((NC * NPAD,), jnp.float32),
        ],
        mesh=plsc.VectorSubcoreMesh(core_axis_name="c", subcore_axis_name="s"),
        scratch_types=[
            pltpu.VMEM((_BR, 1, K), jnp.int32),
            pltpu.VMEM((_BR, 1, K), jnp.int32),
            pltpu.VMEM((_BR, 1, K), jnp.int32),
            pltpu.VMEM((_BR, 1, K), jnp.int32),
            pltpu.VMEM((2, 1, K), jnp.int32),
            pltpu.VMEM((2, 1, K), jnp.int32),
            pltpu.VMEM((K, F), jnp.float32),
            pltpu.VMEM((K, F), jnp.float32),
            pltpu.VMEM((K,), jnp.float32),
            pltpu.VMEM((K,), jnp.float32),
            pltpu.VMEM((K,), jnp.float32),
            pltpu.VMEM((K,), jnp.float32),
            pltpu.VMEM((SL,), jnp.float32),
            pltpu.MemorySpace.VMEM_SHARED((NPAD, F), jnp.float32),
            pltpu.MemorySpace.VMEM_SHARED((NPAD,), jnp.float32),
            pltpu.SemaphoreType.DMA,
            pltpu.SemaphoreType.DMA,
            pltpu.SemaphoreType.DMA,
            pltpu.SemaphoreType.DMA,
            pltpu.SemaphoreType.DMA,
            pltpu.SemaphoreType.DMA,
            pltpu.SemaphoreType.DMA,
            pltpu.SemaphoreType.DMA,
            pltpu.SemaphoreType.DMA,
            pltpu.SemaphoreType.DMA,
        ],
    )(_agg_body)


BN = 1000
GN = N // BN


def _tc_body(feat, aggp, deg2, cp0, cp1, w1s, w1n, w2s, w2n, wfc, out, s0, s1):
    i = pl.program_id(0)

    @pl.when(i == 0)
    def _():
        s0[...] = jnp.zeros_like(s0)
        s1[...] = jnp.zeros_like(s1)

    agg = aggp[0] + aggp[1]
    deg = jnp.maximum(deg2[...], 1.0)
    h = agg / deg
    x1 = jnp.maximum(
        jnp.dot(feat[...], w1s[...], preferred_element_type=jnp.float32)
        + jnp.dot(h, w1n[...], preferred_element_type=jnp.float32), 0.0)
    cv = cp0[...] + cp1[...]
    s0[...] += jnp.sum(x1, axis=0, keepdims=True)
    s1[...] += jnp.sum(x1 * cv, axis=0, keepdims=True)

    @pl.when(i == GN - 1)
    def _():
        g = (jnp.dot(s0[...], w2s[...], preferred_element_type=jnp.float32)
             + jnp.dot(s1[...], w2n[...], preferred_element_type=jnp.float32)
             ) * (1.0 / N)
        out[...] = jax.nn.sigmoid(
            jnp.dot(g, wfc[...], preferred_element_type=jnp.float32))


_tc_kernel = pl.pallas_call(
    _tc_body,
    grid=(GN,),
    in_specs=[
        pl.BlockSpec((BN, F), lambda i: (i, 0)),
        pl.BlockSpec((NC, BN, F), lambda i: (0, i, 0)),
        pl.BlockSpec((BN, 1), lambda i: (i, 0)),
        pl.BlockSpec((BN, 1), lambda i: (i, 0)),
        pl.BlockSpec((BN, 1), lambda i: (i, 0)),
        pl.BlockSpec((F, D), lambda i: (0, 0)),
        pl.BlockSpec((F, D), lambda i: (0, 0)),
        pl.BlockSpec((D, D), lambda i: (0, 0)),
        pl.BlockSpec((D, D), lambda i: (0, 0)),
        pl.BlockSpec((D, 1), lambda i: (0, 0)),
    ],
    out_specs=pl.BlockSpec((1, 1), lambda i: (0, 0)),
    out_shape=jax.ShapeDtypeStruct((1, 1), jnp.float32),
    scratch_shapes=[
        pltpu.VMEM((1, D), jnp.float32),
        pltpu.VMEM((1, D), jnp.float32),
    ],
)


def kernel(features, edge_index, W1_self, W1_neigh, W2_self, W2_neigh, W_fc):
    ed4 = edge_index.reshape(2, ER, 1, K)
    zvec = jnp.zeros((SL,), jnp.float32)
    zrows = jnp.zeros((K, F), jnp.float32)

    deg = _deg_kernel()(ed4, zvec)
    aggp, cp = _agg_kernel()(ed4, features, deg, zvec, zrows)

    deg2 = deg.reshape(NPAD, 1)
    cp0 = cp[:NPAD].reshape(NPAD, 1)
    cp1 = cp[NPAD:].reshape(NPAD, 1)
    return _tc_kernel(features, aggp, deg2, cp0, cp1,
                      W1_self, W1_neigh, W2_self, W2_neigh, W_fc)


# speedup vs baseline: 1.0548x; 1.0548x over previous
import functools

import jax
import jax.numpy as jnp
from jax import lax
from jax.experimental import pallas as pl
from jax.experimental.pallas import tpu as pltpu
from jax.experimental.pallas import tpu_sc as plsc

N = 10000
E = 320000
F = 128
D = 128

NC = 2
NS = 16
K = 128
ER = E // K
NPAD = 10240
SL = NPAD // NS


_ROWS1 = ER // NS
_EXTRA1 = ER - _ROWS1 * NS
_FD = 12


def _deg_body(ed_hbm, zvec_hbm, deg_out, idxb, ones_v, zv, deg_sh, sem):
    sid = lax.axis_index("s")

    pltpu.sync_copy(zvec_hbm, zv)
    pltpu.sync_copy(zv, deg_sh.at[pl.ds(sid * SL, SL)])

    for i in range(K // 16):
        ones_v[pl.ds(i * 16, 16)] = jnp.ones((16,), jnp.float32)

    pltpu.sync_copy(ed_hbm.at[1, pl.ds(sid * _ROWS1, _ROWS1)],
                    idxb.at[pl.ds(0, _ROWS1)])

    @pl.when(sid < _EXTRA1 // 2)
    def _():
        pltpu.sync_copy(ed_hbm.at[1, pl.ds(ER - _EXTRA1 + 2 * sid, 2)],
                        idxb.at[pl.ds(_ROWS1, 2)])

    plsc.subcore_barrier()

    def blk(t, _):
        descs = [
            pltpu.async_copy(ones_v, deg_sh.at[idxb.at[t * _FD + j, 0]], sem,
                             add=True)
            for j in range(_FD)
        ]
        for d in descs:
            d.wait()
        return ()

    lax.fori_loop(0, _ROWS1 // _FD, blk, (), unroll=False)

    @pl.when(sid < _EXTRA1 // 2)
    def _():
        pltpu.sync_copy(ones_v, deg_sh.at[idxb.at[_ROWS1, 0]], add=True)
        pltpu.sync_copy(ones_v, deg_sh.at[idxb.at[_ROWS1 + 1, 0]], add=True)

    plsc.subcore_barrier()

    pltpu.sync_copy(deg_sh.at[pl.ds(sid * SL, SL)], zv)
    pltpu.sync_copy(zv, deg_out.at[pl.ds(sid * SL, SL)])


@functools.cache
def _deg_kernel():
    return functools.partial(
        pl.kernel,
        out_type=jax.ShapeDtypeStruct((NPAD,), jnp.float32),
        mesh=plsc.VectorSubcoreMesh(
            core_axis_name="c", subcore_axis_name="s", num_cores=1),
        scratch_types=[
            pltpu.VMEM((_ROWS1 + 2, 1, K), jnp.int32),
            pltpu.VMEM((K,), jnp.float32),
            pltpu.VMEM((SL,), jnp.float32),
            pltpu.MemorySpace.VMEM_SHARED((NPAD,), jnp.float32),
            pltpu.SemaphoreType.DMA,
        ],
    )(_deg_body)


NW = NC * NS
_ROWS2 = ER // NW
_EXTRA2 = ER - _ROWS2 * NW
_BR = 26
_NB = _ROWS2 // _BR
_BP = _BR // 2


def _agg_body(ed_hbm, feat_hbm, deg_hbm, zvec_hbm, zrows_hbm,
              agg_out, c_out, idxs0, idxd0, idxs1, idxd1, idxsx, idxdx,
              rows0, rows1, degv0, degv1, wv0, wv1, zv, agg_sh, c_sh,
              gsem0, gsem1, dsem0, dsem1, ssem0, ssem1, csem0, csem1,
              ibsem0, ibsem1):
    cid = lax.axis_index("c")
    sid = lax.axis_index("s")
    wid = sid * NC + cid
    base = wid * _ROWS2

    pltpu.sync_copy(zrows_hbm, rows0)
    for k in range(SL // K):
        pltpu.sync_copy(rows0, agg_sh.at[pl.ds(sid * SL + k * K, K)])
    pltpu.sync_copy(zvec_hbm, zv)
    pltpu.sync_copy(zv, c_sh.at[pl.ds(sid * SL, SL)])

    pltpu.sync_copy(ed_hbm.at[0, pl.ds(base, _BR)], idxs0)
    pltpu.sync_copy(ed_hbm.at[1, pl.ds(base, _BR)], idxd0)
    pltpu.async_copy(ed_hbm.at[0, pl.ds(base + _BR, _BR)], idxs1, ibsem1)
    pltpu.async_copy(ed_hbm.at[1, pl.ds(base + _BR, _BR)], idxd1, ibsem1)

    @pl.when(wid < _EXTRA2 // 2)
    def _():
        pltpu.sync_copy(ed_hbm.at[0, pl.ds(ER - _EXTRA2 + 2 * wid, 2)], idxsx)
        pltpu.sync_copy(ed_hbm.at[1, pl.ds(ER - _EXTRA2 + 2 * wid, 2)], idxdx)

    plsc.subcore_barrier()

    def recip(degv, wv):
        for i in range(K // 16):
            dv = degv[pl.ds(i * 16, 16)]
            wv[pl.ds(i * 16, 16)] = 1.0 / jnp.maximum(dv, 1.0)

    def wait_g(rows_v, gsem):
        pltpu.make_async_copy(feat_hbm.at[pl.ds(0, K)], rows_v, gsem).wait()

    def wait_d(degv, dsem):
        pltpu.make_async_copy(deg_hbm.at[pl.ds(0, K)], degv, dsem).wait()

    def wait_s(rows_v, ssem):
        pltpu.make_async_copy(rows_v, agg_sh.at[pl.ds(0, K)], ssem).wait()

    def wait_c(wv, csem):
        pltpu.make_async_copy(wv, c_sh.at[pl.ds(0, K)], csem).wait()

    def wait_ib(idxs_b, idxd_b, ibsem):
        pltpu.make_async_copy(ed_hbm.at[0, pl.ds(0, _BR)], idxs_b, ibsem).wait()
        pltpu.make_async_copy(ed_hbm.at[1, pl.ds(0, _BR)], idxd_b, ibsem).wait()

    def run_block(idxs_b, idxd_b):
        def g_start(slot, rows_v, gsem, degv, dsem):
            pltpu.async_copy(feat_hbm.at[idxs_b.at[slot, 0]], rows_v, gsem)
            pltpu.async_copy(deg_hbm.at[idxd_b.at[slot, 0]], degv, dsem)

        g_start(0, rows0, gsem0, degv0, dsem0)

        def step(t, _):
            a = 2 * t
            b = a + 1
            wait_g(rows0, gsem0)

            @pl.when(t > 0)
            def _():
                wait_s(rows1, ssem1)

            g_start(b, rows1, gsem1, degv1, dsem1)
            pltpu.async_copy(rows0, agg_sh.at[idxd_b.at[a, 0]], ssem0,
                             add=True)
            wait_d(degv0, dsem0)

            @pl.when(t > 0)
            def _():
                wait_c(wv0, csem0)

            recip(degv0, wv0)
            pltpu.async_copy(wv0, c_sh.at[idxs_b.at[a, 0]], csem0, add=True)

            wait_g(rows1, gsem1)
            wait_s(rows0, ssem0)

            @pl.when(t < _BP - 1)
            def _():
                g_start(a + 2, rows0, gsem0, degv0, dsem0)

            pltpu.async_copy(rows1, agg_sh.at[idxd_b.at[b, 0]], ssem1,
                             add=True)
            wait_d(degv1, dsem1)

            @pl.when(t > 0)
            def _():
                wait_c(wv1, csem1)

            recip(degv1, wv1)
            pltpu.async_copy(wv1, c_sh.at[idxs_b.at[b, 0]], csem1, add=True)
            return ()

        lax.fori_loop(0, _BP, step, (), unroll=False)
        wait_s(rows1, ssem1)
        wait_c(wv0, csem0)
        wait_c(wv1, csem1)

    run_block(idxs0, idxd0)
    pltpu.async_copy(ed_hbm.at[0, pl.ds(base + 2 * _BR, _BR)], idxs0, ibsem0)
    pltpu.async_copy(ed_hbm.at[1, pl.ds(base + 2 * _BR, _BR)], idxd0, ibsem0)
    wait_ib(idxs1, idxd1, ibsem1)
    run_block(idxs1, idxd1)
    wait_ib(idxs0, idxd0, ibsem0)
    run_block(idxs0, idxd0)

    @pl.when(wid < _EXTRA2 // 2)
    def _():
        for r in range(2):
            pltpu.async_copy(feat_hbm.at[idxsx.at[r, 0]], rows0, gsem0)
            pltpu.async_copy(deg_hbm.at[idxdx.at[r, 0]], degv0, dsem0)
            wait_g(rows0, gsem0)
            pltpu.sync_copy(rows0, agg_sh.at[idxdx.at[r, 0]], add=True)
            wait_d(degv0, dsem0)
            recip(degv0, wv0)
            pltpu.sync_copy(wv0, c_sh.at[idxsx.at[r, 0]], add=True)

    plsc.subcore_barrier()

    for k in range(SL // K):
        pltpu.sync_copy(agg_sh.at[pl.ds(sid * SL + k * K, K)], rows0)
        pltpu.sync_copy(rows0, agg_out.at[cid, pl.ds(sid * SL + k * K, K)])
    pltpu.sync_copy(c_sh.at[pl.ds(sid * SL, SL)], zv)
    pltpu.sync_copy(zv, c_out.at[pl.ds(cid * NPAD + sid * SL, SL)])


@functools.cache
def _agg_kernel():
    return functools.partial(
        pl.kernel,
        out_type=[
            jax.ShapeDtypeStruct((NC, NPAD, F), jnp.float32),
            jax.ShapeDtypeStruct((NC * NPAD,), jnp.float32),
        ],
        mesh=plsc.VectorSubcoreMesh(core_axis_name="c", subcore_axis_name="s"),
        scratch_types=[
            pltpu.VMEM((_BR, 1, K), jnp.int32),
            pltpu.VMEM((_BR, 1, K), jnp.int32),
            pltpu.VMEM((_BR, 1, K), jnp.int32),
            pltpu.VMEM((_BR, 1, K), jnp.int32),
            pltpu.VMEM((2, 1, K), jnp.int32),
            pltpu.VMEM((2, 1, K), jnp.int32),
            pltpu.VMEM((K, F), jnp.float32),
            pltpu.VMEM((K, F), jnp.float32),
            pltpu.VMEM((K,), jnp.float32),
            pltpu.VMEM((K,), jnp.float32),
            pltpu.VMEM((K,), jnp.float32),
            pltpu.VMEM((K,), jnp.float32),
            pltpu.VMEM((SL,), jnp.float32),
            pltpu.MemorySpace.VMEM_SHARED((NPAD, F), jnp.float32),
            pltpu.MemorySpace.VMEM_SHARED((NPAD,), jnp.float32),
            pltpu.SemaphoreType.DMA,
            pltpu.SemaphoreType.DMA,
            pltpu.SemaphoreType.DMA,
            pltpu.SemaphoreType.DMA,
            pltpu.SemaphoreType.DMA,
            pltpu.SemaphoreType.DMA,
            pltpu.SemaphoreType.DMA,
            pltpu.SemaphoreType.DMA,
            pltpu.SemaphoreType.DMA,
            pltpu.SemaphoreType.DMA,
        ],
    )(_agg_body)


BN = 1024
GN = NPAD // BN


def _tc_body(feat, aggp, deg2, cp0, cp1, w1s, w1n, w2s, w2n, wfc, out, s0, s1):
    i = pl.program_id(0)

    @pl.when(i == 0)
    def _():
        s0[...] = jnp.zeros_like(s0)
        s1[...] = jnp.zeros_like(s1)

    agg = aggp[0] + aggp[1]
    deg = jnp.maximum(deg2[...], 1.0)
    h = agg / deg
    x1 = jnp.maximum(
        jnp.dot(feat[...], w1s[...], preferred_element_type=jnp.float32)
        + jnp.dot(h, w1n[...], preferred_element_type=jnp.float32), 0.0)
    cv = cp0[...] + cp1[...]
    s0[...] += jnp.sum(x1, axis=0, keepdims=True)
    s1[...] += jnp.dot(cv, x1, preferred_element_type=jnp.float32)

    @pl.when(i == GN - 1)
    def _():
        g = (jnp.dot(s0[...], w2s[...], preferred_element_type=jnp.float32)
             + jnp.dot(s1[...], w2n[...], preferred_element_type=jnp.float32)
             ) * (1.0 / N)
        out[...] = jax.nn.sigmoid(
            jnp.dot(g, wfc[...], preferred_element_type=jnp.float32))


_tc_kernel = pl.pallas_call(
    _tc_body,
    grid=(GN,),
    in_specs=[
        pl.BlockSpec((BN, F), lambda i: (i, 0)),
        pl.BlockSpec((NC, BN, F), lambda i: (0, i, 0)),
        pl.BlockSpec((BN, 1), lambda i: (i, 0)),
        pl.BlockSpec((1, BN), lambda i: (0, i)),
        pl.BlockSpec((1, BN), lambda i: (0, i)),
        pl.BlockSpec((F, D), lambda i: (0, 0)),
        pl.BlockSpec((F, D), lambda i: (0, 0)),
        pl.BlockSpec((D, D), lambda i: (0, 0)),
        pl.BlockSpec((D, D), lambda i: (0, 0)),
        pl.BlockSpec((D, 1), lambda i: (0, 0)),
    ],
    out_specs=pl.BlockSpec((1, 1), lambda i: (0, 0)),
    out_shape=jax.ShapeDtypeStruct((1, 1), jnp.float32),
    scratch_shapes=[
        pltpu.VMEM((1, D), jnp.float32),
        pltpu.VMEM((1, D), jnp.float32),
    ],
)


def kernel(features, edge_index, W1_self, W1_neigh, W2_self, W2_neigh, W_fc):
    ed4 = edge_index.reshape(2, ER, 1, K)
    zvec = jnp.zeros((SL,), jnp.float32)
    zrows = jnp.zeros((K, F), jnp.float32)

    deg = _deg_kernel()(ed4, zvec)
    aggp, cp = _agg_kernel()(ed4, features, deg, zvec, zrows)

    featp = jnp.zeros((NPAD, F), jnp.float32).at[:N].set(features)
    deg2 = deg.reshape(NPAD, 1)
    cp0 = cp[:NPAD].reshape(1, NPAD)
    cp1 = cp[NPAD:].reshape(1, NPAD)
    return _tc_kernel(featp, aggp, deg2, cp0, cp1,
                      W1_self, W1_neigh, W2_self, W2_neigh, W_fc)


# speedup vs baseline: 1.0735x; 1.0177x over previous
import functools

import jax
import jax.numpy as jnp
from jax import lax
from jax.experimental import pallas as pl
from jax.experimental.pallas import tpu as pltpu
from jax.experimental.pallas import tpu_sc as plsc

N = 10000
E = 320000
F = 128
D = 128

NC = 2
NS = 16
K = 128
ER = E // K
NPAD = 10240
SL = NPAD // NS


_ROWS1 = ER // NS
_EXTRA1 = ER - _ROWS1 * NS
_FD = 12


def _deg_body(dst_hbm, zvec_hbm, deg_out, idxb, ones_v, zv, deg_sh, sem):
    sid = lax.axis_index("s")

    pltpu.sync_copy(zvec_hbm, zv)
    pltpu.sync_copy(zv, deg_sh.at[pl.ds(sid * SL, SL)])

    for i in range(K // 16):
        ones_v[pl.ds(i * 16, 16)] = jnp.ones((16,), jnp.float32)

    pltpu.sync_copy(dst_hbm.at[pl.ds(sid * _ROWS1, _ROWS1)],
                    idxb.at[pl.ds(0, _ROWS1)])

    @pl.when(sid < _EXTRA1 // 2)
    def _():
        pltpu.sync_copy(dst_hbm.at[pl.ds(ER - _EXTRA1 + 2 * sid, 2)],
                        idxb.at[pl.ds(_ROWS1, 2)])

    plsc.subcore_barrier()

    def blk(t, _):
        descs = [
            pltpu.async_copy(ones_v, deg_sh.at[idxb.at[t * _FD + j, 0]], sem,
                             add=True)
            for j in range(_FD)
        ]
        for d in descs:
            d.wait()
        return ()

    lax.fori_loop(0, _ROWS1 // _FD, blk, (), unroll=False)

    @pl.when(sid < _EXTRA1 // 2)
    def _():
        pltpu.sync_copy(ones_v, deg_sh.at[idxb.at[_ROWS1, 0]], add=True)
        pltpu.sync_copy(ones_v, deg_sh.at[idxb.at[_ROWS1 + 1, 0]], add=True)

    plsc.subcore_barrier()

    pltpu.sync_copy(deg_sh.at[pl.ds(sid * SL, SL)], zv)
    pltpu.sync_copy(zv, deg_out.at[pl.ds(sid * SL, SL)])


@functools.cache
def _deg_kernel():
    return functools.partial(
        pl.kernel,
        out_type=jax.ShapeDtypeStruct((NPAD,), jnp.float32),
        mesh=plsc.VectorSubcoreMesh(
            core_axis_name="c", subcore_axis_name="s", num_cores=1),
        scratch_types=[
            pltpu.VMEM((_ROWS1 + 2, 1, K), jnp.int32),
            pltpu.VMEM((K,), jnp.float32),
            pltpu.VMEM((SL,), jnp.float32),
            pltpu.MemorySpace.VMEM_SHARED((NPAD,), jnp.float32),
            pltpu.SemaphoreType.DMA,
        ],
    )(_deg_body)


NW = NC * NS
_ROWS2 = ER // NW
_EXTRA2 = ER - _ROWS2 * NW
_BR = 26
_NB = _ROWS2 // _BR
_BP = _BR // 2


def _agg_body(src_hbm, dst_hbm, feat_hbm, deg_hbm, zvec_hbm, zrows_hbm,
              agg_out, c_out, idxs0, idxd0, idxs1, idxd1, idxsx, idxdx,
              rows0, rows1, degv0, degv1, wv0, wv1, zv, agg_sh, c_sh,
              gsem0, gsem1, dsem0, dsem1, ssem0, ssem1, csem0, csem1,
              ibsem0, ibsem1):
    cid = lax.axis_index("c")
    sid = lax.axis_index("s")
    wid = sid * NC + cid
    base = wid * _ROWS2

    pltpu.sync_copy(src_hbm.at[pl.ds(base, _BR)], idxs0)
    pltpu.sync_copy(dst_hbm.at[pl.ds(base, _BR)], idxd0)
    pltpu.async_copy(feat_hbm.at[idxs0.at[0, 0]], rows0, gsem0)
    pltpu.async_copy(deg_hbm.at[idxd0.at[0, 0]], degv0, dsem0)
    pltpu.async_copy(src_hbm.at[pl.ds(base + _BR, _BR)], idxs1, ibsem1)
    pltpu.async_copy(dst_hbm.at[pl.ds(base + _BR, _BR)], idxd1, ibsem1)

    @pl.when(wid < _EXTRA2 // 2)
    def _():
        pltpu.sync_copy(src_hbm.at[pl.ds(ER - _EXTRA2 + 2 * wid, 2)], idxsx)
        pltpu.sync_copy(dst_hbm.at[pl.ds(ER - _EXTRA2 + 2 * wid, 2)], idxdx)

    pltpu.sync_copy(zrows_hbm, rows1)
    for k in range(SL // K):
        pltpu.sync_copy(rows1, agg_sh.at[pl.ds(sid * SL + k * K, K)])
    pltpu.sync_copy(zvec_hbm, zv)
    pltpu.sync_copy(zv, c_sh.at[pl.ds(sid * SL, SL)])

    plsc.subcore_barrier()

    def recip(degv, wv):
        for i in range(K // 16):
            dv = degv[pl.ds(i * 16, 16)]
            wv[pl.ds(i * 16, 16)] = 1.0 / jnp.maximum(dv, 1.0)

    def wait_g(rows_v, gsem):
        pltpu.make_async_copy(feat_hbm.at[pl.ds(0, K)], rows_v, gsem).wait()

    def wait_d(degv, dsem):
        pltpu.make_async_copy(deg_hbm.at[pl.ds(0, K)], degv, dsem).wait()

    def wait_s(rows_v, ssem):
        pltpu.make_async_copy(rows_v, agg_sh.at[pl.ds(0, K)], ssem).wait()

    def wait_c(wv, csem):
        pltpu.make_async_copy(wv, c_sh.at[pl.ds(0, K)], csem).wait()

    def wait_ib(idxs_b, idxd_b, ibsem):
        pltpu.make_async_copy(src_hbm.at[pl.ds(0, _BR)], idxs_b, ibsem).wait()
        pltpu.make_async_copy(dst_hbm.at[pl.ds(0, _BR)], idxd_b, ibsem).wait()

    def run_block(idxs_b, idxd_b, prologue=True):
        def g_start(slot, rows_v, gsem, degv, dsem):
            pltpu.async_copy(feat_hbm.at[idxs_b.at[slot, 0]], rows_v, gsem)
            pltpu.async_copy(deg_hbm.at[idxd_b.at[slot, 0]], degv, dsem)

        if prologue:
            g_start(0, rows0, gsem0, degv0, dsem0)

        def step(t, _):
            a = 2 * t
            b = a + 1
            wait_g(rows0, gsem0)

            @pl.when(t > 0)
            def _():
                wait_s(rows1, ssem1)

            g_start(b, rows1, gsem1, degv1, dsem1)
            pltpu.async_copy(rows0, agg_sh.at[idxd_b.at[a, 0]], ssem0,
                             add=True)
            wait_d(degv0, dsem0)

            @pl.when(t > 0)
            def _():
                wait_c(wv0, csem0)

            recip(degv0, wv0)
            pltpu.async_copy(wv0, c_sh.at[idxs_b.at[a, 0]], csem0, add=True)

            wait_g(rows1, gsem1)
            wait_s(rows0, ssem0)

            @pl.when(t < _BP - 1)
            def _():
                g_start(a + 2, rows0, gsem0, degv0, dsem0)

            pltpu.async_copy(rows1, agg_sh.at[idxd_b.at[b, 0]], ssem1,
                             add=True)
            wait_d(degv1, dsem1)

            @pl.when(t > 0)
            def _():
                wait_c(wv1, csem1)

            recip(degv1, wv1)
            pltpu.async_copy(wv1, c_sh.at[idxs_b.at[b, 0]], csem1, add=True)
            return ()

        lax.fori_loop(0, _BP, step, (), unroll=False)
        wait_s(rows1, ssem1)
        wait_c(wv0, csem0)
        wait_c(wv1, csem1)

    run_block(idxs0, idxd0, prologue=False)
    pltpu.async_copy(src_hbm.at[pl.ds(base + 2 * _BR, _BR)], idxs0, ibsem0)
    pltpu.async_copy(dst_hbm.at[pl.ds(base + 2 * _BR, _BR)], idxd0, ibsem0)
    wait_ib(idxs1, idxd1, ibsem1)
    run_block(idxs1, idxd1)
    wait_ib(idxs0, idxd0, ibsem0)
    run_block(idxs0, idxd0)

    @pl.when(wid < _EXTRA2 // 2)
    def _():
        for r in range(2):
            pltpu.async_copy(feat_hbm.at[idxsx.at[r, 0]], rows0, gsem0)
            pltpu.async_copy(deg_hbm.at[idxdx.at[r, 0]], degv0, dsem0)
            wait_g(rows0, gsem0)
            pltpu.sync_copy(rows0, agg_sh.at[idxdx.at[r, 0]], add=True)
            wait_d(degv0, dsem0)
            recip(degv0, wv0)
            pltpu.sync_copy(wv0, c_sh.at[idxsx.at[r, 0]], add=True)

    plsc.subcore_barrier()

    for k in range(SL // K):
        pltpu.sync_copy(agg_sh.at[pl.ds(sid * SL + k * K, K)], rows0)
        pltpu.sync_copy(rows0, agg_out.at[cid, pl.ds(sid * SL + k * K, K)])
    pltpu.sync_copy(c_sh.at[pl.ds(sid * SL, SL)], zv)
    pltpu.sync_copy(zv, c_out.at[pl.ds(cid * NPAD + sid * SL, SL)])


@functools.cache
def _agg_kernel():
    return functools.partial(
        pl.kernel,
        out_type=[
            jax.ShapeDtypeStruct((NC, NPAD, F), jnp.float32),
            jax.ShapeDtypeStruct((NC * NPAD,), jnp.float32),
        ],
        mesh=plsc.VectorSubcoreMesh(core_axis_name="c", subcore_axis_name="s"),
        scratch_types=[
            pltpu.VMEM((_BR, 1, K), jnp.int32),
            pltpu.VMEM((_BR, 1, K), jnp.int32),
            pltpu.VMEM((_BR, 1, K), jnp.int32),
            pltpu.VMEM((_BR, 1, K), jnp.int32),
            pltpu.VMEM((2, 1, K), jnp.int32),
            pltpu.VMEM((2, 1, K), jnp.int32),
            pltpu.VMEM((K, F), jnp.float32),
            pltpu.VMEM((K, F), jnp.float32),
            pltpu.VMEM((K,), jnp.float32),
            pltpu.VMEM((K,), jnp.float32),
            pltpu.VMEM((K,), jnp.float32),
            pltpu.VMEM((K,), jnp.float32),
            pltpu.VMEM((SL,), jnp.float32),
            pltpu.MemorySpace.VMEM_SHARED((NPAD, F), jnp.float32),
            pltpu.MemorySpace.VMEM_SHARED((NPAD,), jnp.float32),
            pltpu.SemaphoreType.DMA,
            pltpu.SemaphoreType.DMA,
            pltpu.SemaphoreType.DMA,
            pltpu.SemaphoreType.DMA,
            pltpu.SemaphoreType.DMA,
            pltpu.SemaphoreType.DMA,
            pltpu.SemaphoreType.DMA,
            pltpu.SemaphoreType.DMA,
            pltpu.SemaphoreType.DMA,
            pltpu.SemaphoreType.DMA,
        ],
    )(_agg_body)


BN = 1024
GN = NPAD // BN


def _tc_body(feat, aggp, deg2, cp0, cp1, w1s, w1n, w2s, w2n, wfc, out, s0, s1):
    i = pl.program_id(0)

    @pl.when(i == 0)
    def _():
        s0[...] = jnp.zeros_like(s0)
        s1[...] = jnp.zeros_like(s1)

    agg = aggp[0] + aggp[1]
    deg = jnp.maximum(deg2[...], 1.0)
    h = agg / deg
    x1 = jnp.maximum(
        jnp.dot(feat[...], w1s[...], preferred_element_type=jnp.float32)
        + jnp.dot(h, w1n[...], preferred_element_type=jnp.float32), 0.0)
    cv = cp0[...] + cp1[...]
    s0[...] += jnp.sum(x1, axis=0, keepdims=True)
    s1[...] += jnp.dot(cv, x1, preferred_element_type=jnp.float32)

    @pl.when(i == GN - 1)
    def _():
        g = (jnp.dot(s0[...], w2s[...], preferred_element_type=jnp.float32)
             + jnp.dot(s1[...], w2n[...], preferred_element_type=jnp.float32)
             ) * (1.0 / N)
        out[...] = jax.nn.sigmoid(
            jnp.dot(g, wfc[...], preferred_element_type=jnp.float32))


_tc_kernel = pl.pallas_call(
    _tc_body,
    grid=(GN,),
    in_specs=[
        pl.BlockSpec((BN, F), lambda i: (i, 0)),
        pl.BlockSpec((NC, BN, F), lambda i: (0, i, 0)),
        pl.BlockSpec((BN, 1), lambda i: (i, 0)),
        pl.BlockSpec((1, BN), lambda i: (0, i)),
        pl.BlockSpec((1, BN), lambda i: (0, i)),
        pl.BlockSpec((F, D), lambda i: (0, 0)),
        pl.BlockSpec((F, D), lambda i: (0, 0)),
        pl.BlockSpec((D, D), lambda i: (0, 0)),
        pl.BlockSpec((D, D), lambda i: (0, 0)),
        pl.BlockSpec((D, 1), lambda i: (0, 0)),
    ],
    out_specs=pl.BlockSpec((1, 1), lambda i: (0, 0)),
    out_shape=jax.ShapeDtypeStruct((1, 1), jnp.float32),
    scratch_shapes=[
        pltpu.VMEM((1, D), jnp.float32),
        pltpu.VMEM((1, D), jnp.float32),
    ],
)


def kernel(features, edge_index, W1_self, W1_neigh, W2_self, W2_neigh, W_fc):
    src2 = edge_index[0].reshape(ER, 1, K)
    dst2 = edge_index[1].reshape(ER, 1, K)
    zvec = jnp.zeros((SL,), jnp.float32)
    zrows = jnp.zeros((K, F), jnp.float32)

    deg = _deg_kernel()(dst2, zvec)
    aggp, cp = _agg_kernel()(src2, dst2, features, deg, zvec, zrows)

    featp = jnp.zeros((NPAD, F), jnp.float32).at[:N].set(features)
    deg2 = deg.reshape(NPAD, 1)
    cp0 = cp[:NPAD].reshape(1, NPAD)
    cp1 = cp[NPAD:].reshape(1, NPAD)
    return _tc_kernel(featp, aggp, deg2, cp0, cp1,
                      W1_self, W1_neigh, W2_self, W2_neigh, W_fc)
